# baseline (speedup 1.0000x reference)
import functools

import jax
import jax.numpy as jnp
from jax import lax
from jax.experimental import pallas as pl
from jax.experimental.pallas import tpu as pltpu
from jax.experimental.pallas import tpu_sc as plsc

_L = 16
_ROWS = 8192
_C = 2048
_GCOLS = 128
_NVEC = _GCOLS // _L
_HROWS = _ROWS // 2
_CH = 512
_NCHUNK = _HROWS // _CH


def _sc_body(x_hbm, o_hbm, buf, sums, psums, shared):
    c = lax.axis_index("c")
    s = lax.axis_index("s")
    cb = (c * 8 + lax.rem(s, 8)) * _GCOLS
    rb = (s // 8) * _HROWS

    def chunk1(g, acc):
        pltpu.sync_copy(
            x_hbm.at[pl.ds(rb + g * _CH, _CH), pl.ds(cb, _GCOLS)], buf
        )

        def row(r, acc):
            return tuple(
                acc[k] + buf[r, pl.ds(k * _L, _L)] for k in range(_NVEC)
            )

        return lax.fori_loop(0, _CH, row, acc)

    zero = jnp.zeros((_L,), jnp.float32)
    acc = lax.fori_loop(0, _NCHUNK, chunk1, (zero,) * _NVEC)

    for k in range(_NVEC):
        sums[pl.ds(k * _L, _L)] = acc[k]
    pltpu.sync_copy(sums, shared.at[s])
    plsc.subcore_barrier()
    pltpu.sync_copy(shared.at[s ^ 8], psums)

    inv_n = 1.0 / _ROWS
    parity = lax.rem(lax.iota(jnp.int32, _L), 2) == 0
    means = [
        (acc[k] + psums[pl.ds(k * _L, _L)]) * inv_n for k in range(_NVEC)
    ]

    def chunk2(g, carry):
        pltpu.sync_copy(
            x_hbm.at[pl.ds(rb + g * _CH, _CH), pl.ds(cb, _GCOLS)], buf
        )

        def row(r, carry):
            for k in range(_NVEC):
                v = buf[r, pl.ds(k * _L, _L)]
                buf[r, pl.ds(k * _L, _L)] = jnp.where(parity, means[k], v)
            return carry

        lax.fori_loop(0, _CH, row, 0)
        pltpu.sync_copy(
            buf, o_hbm.at[pl.ds(rb + g * _CH, _CH), pl.ds(cb, _GCOLS)]
        )
        return carry

    lax.fori_loop(0, _NCHUNK, chunk2, 0)


def kernel(inputs):
    orig_shape = inputs.shape
    x = inputs.reshape(_ROWS, _C)
    mesh = plsc.VectorSubcoreMesh(core_axis_name="c", subcore_axis_name="s")
    sc_call = functools.partial(
        pl.kernel,
        mesh=mesh,
        out_type=jax.ShapeDtypeStruct((_ROWS, _C), jnp.float32),
        scratch_types=[
            pltpu.VMEM((_CH, _GCOLS), jnp.float32),
            pltpu.VMEM((_GCOLS,), jnp.float32),
            pltpu.VMEM((_GCOLS,), jnp.float32),
            pltpu.VMEM_SHARED((16, _GCOLS), jnp.float32),
        ],
    )(_sc_body)
    out = sc_call(x)
    return out.reshape(orig_shape)


# speedup vs baseline: 3.0694x; 3.0694x over previous
import functools

import jax
import jax.numpy as jnp
from jax.experimental import pallas as pl

_STRIPE_W = 256


def _stripe_kernel(x_ref, o_ref, *, inv_n):
    x = x_ref[...]
    mean = jnp.sum(x, axis=0, keepdims=True) * inv_n
    lane = jax.lax.broadcasted_iota(jnp.int32, x.shape, dimension=1)
    o_ref[...] = jnp.where(lane % 2 == 0, jnp.broadcast_to(mean, x.shape), x)


def kernel(inputs):
    orig_shape = inputs.shape
    c = orig_shape[-1]
    rows = 1
    for d in orig_shape[:-1]:
        rows *= d
    x = inputs.reshape(rows, c)
    nstripes = c // _STRIPE_W

    out = pl.pallas_call(
        functools.partial(_stripe_kernel, inv_n=1.0 / rows),
        grid=(nstripes,),
        in_specs=[pl.BlockSpec((rows, _STRIPE_W), lambda s: (0, s))],
        out_specs=pl.BlockSpec((rows, _STRIPE_W), lambda s: (0, s)),
        out_shape=jax.ShapeDtypeStruct((rows, c), jnp.float32),
    )(x)

    return out.reshape(orig_shape)
